# Initial kernel scaffold; baseline (speedup 1.0000x reference)
import jax
import jax.numpy as jnp
from jax.experimental import pallas as pl


def kernel(input_ids, table):
    raise NotImplementedError("write your pallas kernel here")


# speedup vs baseline: 2.9815x; 2.9815x over previous
import functools

import jax
import jax.numpy as jnp
from jax import lax
from jax.experimental import pallas as pl
from jax.experimental.pallas import tpu as pltpu
from jax.experimental.pallas import tpu_sc as plsc

D_E = 128
NUM_WORKERS = 32
CHUNK = 128


def _sc_gather(idx2d, table, n_chunks):
    mesh = plsc.VectorSubcoreMesh(core_axis_name="c", subcore_axis_name="s")

    @functools.partial(
        pl.kernel,
        out_type=jax.ShapeDtypeStruct(
            (NUM_WORKERS, n_chunks, CHUNK, D_E), jnp.float32
        ),
        mesh=mesh,
        scratch_types=[
            pltpu.VMEM((n_chunks, CHUNK), jnp.int32),
            pltpu.VMEM((CHUNK, D_E), jnp.float32),
            pltpu.SemaphoreType.DMA,
        ],
    )
    def k(idx_hbm, table_hbm, out_hbm, idx_v, rows_v, gsem):
        wid = lax.axis_index("s") * 2 + lax.axis_index("c")
        pltpu.sync_copy(idx_hbm.at[wid], idx_v)

        def body(j, _):
            pltpu.async_copy(table_hbm.at[idx_v.at[j]], rows_v, gsem).wait()
            pltpu.sync_copy(rows_v, out_hbm.at[wid].at[j])
            return 0

        lax.fori_loop(0, n_chunks, body, 0, unroll=False)

    return k(idx2d, table)


def kernel(input_ids, table):
    b, s = input_ids.shape
    total = b * s
    n_chunks = total // (NUM_WORKERS * CHUNK)
    assert n_chunks * NUM_WORKERS * CHUNK == total
    idx2d = input_ids.reshape(NUM_WORKERS, n_chunks, CHUNK).astype(jnp.int32)
    out = _sc_gather(idx2d, table, n_chunks)
    return out.reshape(b, s, D_E)


# speedup vs baseline: 3.3365x; 1.1191x over previous
import functools

import jax
import jax.numpy as jnp
from jax import lax
from jax.experimental import pallas as pl
from jax.experimental.pallas import tpu as pltpu
from jax.experimental.pallas import tpu_sc as plsc

D_E = 128
NUM_WORKERS = 32
CHUNK = 128


def _sc_gather(idx2d, table, n_chunks):
    mesh = plsc.VectorSubcoreMesh(core_axis_name="c", subcore_axis_name="s")

    @functools.partial(
        pl.kernel,
        out_type=jax.ShapeDtypeStruct(
            (NUM_WORKERS, n_chunks, CHUNK, D_E), jnp.float32
        ),
        mesh=mesh,
        scratch_types=[
            pltpu.VMEM((n_chunks, CHUNK), jnp.int32),
            pltpu.VMEM((2, CHUNK, D_E), jnp.float32),
            pltpu.SemaphoreType.DMA,
            pltpu.SemaphoreType.DMA,
        ],
    )
    def k(idx_hbm, table_hbm, out_hbm, idx_v, rows_v, g0, g1):
        assert n_chunks % 2 == 0
        wid = lax.axis_index("s") * 2 + lax.axis_index("c")
        pltpu.sync_copy(idx_hbm.at[wid], idx_v)

        pltpu.async_copy(table_hbm.at[idx_v.at[0]], rows_v.at[0], g0)

        def body(j2, _):
            j = j2 * 2
            pltpu.async_copy(table_hbm.at[idx_v.at[j + 1]], rows_v.at[1], g1)
            pltpu.make_async_copy(
                table_hbm.at[idx_v.at[j]], rows_v.at[0], g0
            ).wait()
            pltpu.sync_copy(rows_v.at[0], out_hbm.at[wid].at[j])

            @pl.when(j + 2 < n_chunks)
            def _():
                pltpu.async_copy(
                    table_hbm.at[idx_v.at[j + 2]], rows_v.at[0], g0
                )

            pltpu.make_async_copy(
                table_hbm.at[idx_v.at[j + 1]], rows_v.at[1], g1
            ).wait()
            pltpu.sync_copy(rows_v.at[1], out_hbm.at[wid].at[j + 1])
            return 0

        lax.fori_loop(0, n_chunks // 2, body, 0, unroll=False)

    return k(idx2d, table)


def kernel(input_ids, table):
    b, s = input_ids.shape
    total = b * s
    n_chunks = total // (NUM_WORKERS * CHUNK)
    assert n_chunks * NUM_WORKERS * CHUNK == total
    idx2d = input_ids.reshape(NUM_WORKERS, n_chunks, CHUNK).astype(jnp.int32)
    out = _sc_gather(idx2d, table, n_chunks)
    return out.reshape(b, s, D_E)


# speedup vs baseline: 5.1321x; 1.5382x over previous
import functools

import jax
import jax.numpy as jnp
from jax import lax
from jax.experimental import pallas as pl
from jax.experimental.pallas import tpu as pltpu
from jax.experimental.pallas import tpu_sc as plsc

D_E = 128
NUM_WORKERS = 32


def _sc_gather(idx3, table, per_w, seq):
    n_batch = NUM_WORKERS * per_w
    mesh = plsc.VectorSubcoreMesh(core_axis_name="c", subcore_axis_name="s")

    @functools.partial(
        pl.kernel,
        out_type=jax.ShapeDtypeStruct((n_batch, seq, D_E), jnp.float32),
        mesh=mesh,
        scratch_types=[
            pltpu.VMEM((per_w, seq), jnp.int32),
            pltpu.VMEM((2, seq, D_E), jnp.float32),
            pltpu.SemaphoreType.DMA,
            pltpu.SemaphoreType.DMA,
        ],
    )
    def k(idx_hbm, table_hbm, out_hbm, idx_v, rows_v, g0, g1):
        assert per_w % 2 == 0
        wid = lax.axis_index("s") * 2 + lax.axis_index("c")
        base = wid * per_w
        pltpu.sync_copy(idx_hbm.at[wid], idx_v)

        pltpu.async_copy(table_hbm.at[idx_v.at[0]], rows_v.at[0], g0)

        def body(i, _):
            b = i * 2
            pltpu.async_copy(table_hbm.at[idx_v.at[b + 1]], rows_v.at[1], g1)
            pltpu.make_async_copy(
                table_hbm.at[idx_v.at[b]], rows_v.at[0], g0
            ).wait()
            pltpu.sync_copy(rows_v.at[0], out_hbm.at[base + b])

            @pl.when(b + 2 < per_w)
            def _():
                pltpu.async_copy(
                    table_hbm.at[idx_v.at[b + 2]], rows_v.at[0], g0
                )

            pltpu.make_async_copy(
                table_hbm.at[idx_v.at[b + 1]], rows_v.at[1], g1
            ).wait()
            pltpu.sync_copy(rows_v.at[1], out_hbm.at[base + b + 1])
            return 0

        lax.fori_loop(0, per_w // 2, body, 0, unroll=False)

    return k(idx3, table)


def kernel(input_ids, table):
    b, s = input_ids.shape
    per_w = b // NUM_WORKERS
    assert per_w * NUM_WORKERS == b
    idx3 = input_ids.reshape(NUM_WORKERS, per_w, s).astype(jnp.int32)
    return _sc_gather(idx3, table, per_w, s)
